# Initial kernel scaffold; baseline (speedup 1.0000x reference)
import jax
import jax.numpy as jnp
from jax.experimental import pallas as pl


def kernel(x, theta):
    raise NotImplementedError("write your pallas kernel here")


# speedup vs baseline: 1.3497x; 1.3497x over previous
import functools

import numpy as np
import jax
import jax.numpy as jnp
from jax import lax
from jax.experimental import pallas as pl
from jax.experimental.pallas import tpu as pltpu
from jax.experimental.pallas import tpu_sc as plsc

B, H, W, C = 4, 384, 384, 96
HW = H * W
N = B * HW
NW = 32
PIX_PER_TILE = N // NW
P = 128
CHUNKS = PIX_PER_TILE // P
CSUB = C // 16
GROUPS = P // 16

_DELTA = float(np.float32(2.0) / np.float32(383.0))


def _roundbf(v):
    u = plsc.bitcast(v, jnp.int32)
    lsb = lax.shift_right_logical(u, 16) & 1
    r = (u + 32767 + lsb) & jnp.int32(-65536)
    return plsc.bitcast(r, jnp.float32)


def _tile_body(xf, th, out, theta_v, idxa_v, idxb_v, idxc_v, idxd_v,
               bufa_v, bufb_v, bufc_v, bufd_v,
               wa_v, wb_v, wc_v, wd_v, out_v, sem):
    cid = lax.axis_index("c")
    sid = lax.axis_index("s")
    wid = sid * 2 + cid
    base = wid * PIX_PER_TILE
    b = base // HW
    b6 = b * 6

    pltpu.sync_copy(th, theta_v)
    tvec = _roundbf(plsc.load_gather(
        theta_v, [b6 + lax.broadcasted_iota(jnp.int32, (16,), 0)]))
    t00 = tvec[0]
    t01 = tvec[1]
    t02 = tvec[2]
    t10 = tvec[3]
    t11 = tvec[4]
    t12 = tvec[5]
    rowb = b * HW

    def chunk(k, carry):
        n0 = base + k * P
        for g in range(GROUPS):
            n = n0 + g * 16 + lax.broadcasted_iota(jnp.int32, (16,), 0)
            w_i = n % W
            h_i = (n // W) % H
            xs = _roundbf(w_i.astype(jnp.float32) * _DELTA - 1.0)
            ys = _roundbf(h_i.astype(jnp.float32) * _DELTA - 1.0)
            gx = t00 * xs + t01 * ys + t02
            gy = t10 * xs + t11 * ys + t12
            px = ((gx + 1.0) * 382.0) * 0.5
            py = ((gy + 1.0) * 382.0) * 0.5
            xt = px.astype(jnp.int32)
            x0 = jnp.where(px < xt.astype(jnp.float32), xt - 1, xt)
            yt = py.astype(jnp.int32)
            y0 = jnp.where(py < yt.astype(jnp.float32), yt - 1, yt)
            x1 = x0 + 1
            y1 = y0 + 1
            x0c = jnp.clip(x0, 0, W - 1)
            x1c = jnp.clip(x1, 0, W - 1)
            y0c = jnp.clip(y0, 0, H - 1)
            y1c = jnp.clip(y1, 0, H - 1)
            x0f = x0c.astype(jnp.float32)
            x1f = x1c.astype(jnp.float32)
            y0f = y0c.astype(jnp.float32)
            y1f = y1c.astype(jnp.float32)
            sl = pl.ds(g * 16, 16)
            wa_v[sl] = (x1f - px) * (y1f - py)
            wb_v[sl] = (x1f - px) * (py - y0f)
            wc_v[sl] = (px - x0f) * (y1f - py)
            wd_v[sl] = (px - x0f) * (py - y0f)
            r0 = rowb + y0c * W
            r1 = rowb + y1c * W
            idxa_v[sl] = r0 + x0c
            idxb_v[sl] = r1 + x0c
            idxc_v[sl] = r0 + x1c
            idxd_v[sl] = r1 + x1c

        ca = pltpu.async_copy(xf.at[idxa_v], bufa_v, sem)
        cb = pltpu.async_copy(xf.at[idxb_v], bufb_v, sem)
        cc = pltpu.async_copy(xf.at[idxc_v], bufc_v, sem)
        cd = pltpu.async_copy(xf.at[idxd_v], bufd_v, sem)
        ca.wait()
        cb.wait()
        cc.wait()
        cd.wait()

        def blend(g, c2):
            gsl = pl.ds(g * 16, 16)
            wavec = wa_v[gsl]
            wbvec = wb_v[gsl]
            wcvec = wc_v[gsl]
            wdvec = wd_v[gsl]
            for j in range(16):
                p = g * 16 + j
                was = wavec[j]
                wbs = wbvec[j]
                wcs = wcvec[j]
                wds = wdvec[j]
                for s in range(CSUB):
                    cs = pl.ds(s * 16, 16)
                    out_v[p, cs] = ((was * bufa_v[p, cs] + wbs * bufb_v[p, cs])
                                    + wcs * bufc_v[p, cs]) + wds * bufd_v[p, cs]
            return c2

        lax.fori_loop(0, GROUPS, blend, 0, unroll=False)

        pltpu.sync_copy(out_v, out.at[pl.ds(n0, P)])
        return carry

    lax.fori_loop(0, CHUNKS, chunk, 0, unroll=False)


@jax.jit
def kernel(x, theta):
    xf = x.reshape(N, C)
    th = jnp.pad(theta.reshape(-1), (0, 40)).astype(jnp.float32)
    mesh = plsc.VectorSubcoreMesh(core_axis_name="c", subcore_axis_name="s")
    run = pl.kernel(
        _tile_body,
        out_type=jax.ShapeDtypeStruct((N, C), jnp.float32),
        mesh=mesh,
        compiler_params=pltpu.CompilerParams(
            use_tc_tiling_on_sc=False,
            needs_layout_passes=False,
        ),
        scratch_types=[
            pltpu.VMEM((64,), jnp.float32),
            pltpu.VMEM((P,), jnp.int32),
            pltpu.VMEM((P,), jnp.int32),
            pltpu.VMEM((P,), jnp.int32),
            pltpu.VMEM((P,), jnp.int32),
            pltpu.VMEM((P, C), jnp.float32),
            pltpu.VMEM((P, C), jnp.float32),
            pltpu.VMEM((P, C), jnp.float32),
            pltpu.VMEM((P, C), jnp.float32),
            pltpu.VMEM((P,), jnp.float32),
            pltpu.VMEM((P,), jnp.float32),
            pltpu.VMEM((P,), jnp.float32),
            pltpu.VMEM((P,), jnp.float32),
            pltpu.VMEM((P, C), jnp.float32),
            pltpu.SemaphoreType.DMA,
        ],
    )
    outf = run(xf, th)
    return outf.reshape(B, H, W, C)
